# baseline (speedup 1.0000x reference)
import jax
import jax.numpy as jnp
from jax import lax
from jax.experimental import pallas as pl
from jax.experimental.pallas import tpu as pltpu
from jax.experimental.pallas import tpu_sc as plsc

E = 6_400_000
EHALF = E // 2
N_ML = 100_000
N_MM = 100_000

NC = 2
NS = 16
LANES = 16
NW = NC * NS
EPT = EHALF // NW
CHUNK = 2000
NCHUNK = EPT // CHUNK
NPAIR = NCHUNK // 2
GROUPS = CHUNK // LANES

ROWS_PER_SUB = 6256
PAD = ROWS_PER_SUB * NS

CUTOFF = 12.0
KE = 332.0637
CUTON = 9.0
CHI_SHIFT = 1.0 / CUTOFF
CHI2_SHIFT = CHI_SHIFT * CHI_SHIFT
INV_W = 1.0 / (CUTOFF - CUTON)


def _sc_body(d_hbm, idxu_hbm, idxv_hbm, vx_hbm, vy_hbm, vz_hbm,
             qdx_hbm, dydz_hbm, qmm_hbm, out_hbm,
             qdx_s, dydz_s, qmm_s,
             d_v, idxu_v, idxv_v, vx_v, vy_v, vz_v,
             g1_v, g2_v, qj_v, out_v,
             sem_in0, sem_in1, sem_g0, sem_g1, sem_o0, sem_o1,
             *, base0):
    c = lax.axis_index("c")
    s = lax.axis_index("s")
    wid = s * NC + c
    tile_base = base0 + wid * EPT

    sl = pl.ds(s * ROWS_PER_SUB, ROWS_PER_SUB)
    bounce_f = vx_v.at[pl.ds(0, ROWS_PER_SUB)]
    pltpu.sync_copy(qmm_hbm.at[sl], bounce_f)
    pltpu.sync_copy(bounce_f, qmm_s.at[sl])
    bounce_i = idxu_v.at[pl.ds(0, ROWS_PER_SUB)]
    for hbm_t, sp_t in ((qdx_hbm, qdx_s), (dydz_hbm, dydz_s)):
        pltpu.sync_copy(hbm_t.at[sl], bounce_i)
        pltpu.sync_copy(bounce_i, sp_t.at[sl])
    plsc.subcore_barrier()

    ins = (d_hbm, idxu_hbm, idxv_hbm, vx_hbm, vy_hbm, vz_hbm)
    inbufs = (d_v, idxu_v, idxv_v, vx_v, vy_v, vz_v)
    sem_in = (sem_in0, sem_in1)
    sem_g = (sem_g0, sem_g1)
    sem_o = (sem_o0, sem_o1)

    def ces(k):
        return pl.ds(tile_base + k * CHUNK, CHUNK)

    def oes(k):
        return pl.ds(tile_base - base0 + k * CHUNK, CHUNK)

    def half(buf, p):
        return buf.at[pl.ds(p * CHUNK, CHUNK)]

    def instream(k, p):
        sli = ces(k)
        for hbm_t, buf in zip(ins, inbufs):
            pltpu.async_copy(hbm_t.at[sli], half(buf, p), sem_in[p])

    def instream_wait(k, p):
        sli = ces(k)
        for hbm_t, buf in zip(ins, inbufs):
            pltpu.make_async_copy(hbm_t.at[sli], half(buf, p), sem_in[p]).wait()

    def gather(p):
        pltpu.async_copy(qdx_s.at[half(idxu_v, p)], half(g1_v, p), sem_g[p])
        pltpu.async_copy(dydz_s.at[half(idxu_v, p)], half(g2_v, p), sem_g[p])
        pltpu.async_copy(qmm_s.at[half(idxv_v, p)], half(qj_v, p), sem_g[p])

    def gather_wait(p):
        pltpu.make_async_copy(qdx_s.at[half(idxu_v, p)], half(g1_v, p),
                              sem_g[p]).wait()
        pltpu.make_async_copy(dydz_s.at[half(idxu_v, p)], half(g2_v, p),
                              sem_g[p]).wait()
        pltpu.make_async_copy(qmm_s.at[half(idxv_v, p)], half(qj_v, p),
                              sem_g[p]).wait()

    def outstream(k, p):
        pltpu.async_copy(half(out_v, p), out_hbm.at[oes(k)], sem_o[p])

    def outstream_wait(k, p):
        pltpu.make_async_copy(half(out_v, p), out_hbm.at[oes(k)],
                              sem_o[p]).wait()

    def compute(p):
        dp, qjp = half(d_v, p), half(qj_v, p)
        g1p, g2p = half(g1_v, p), half(g2_v, p)
        vxp, vyp, vzp = half(vx_v, p), half(vy_v, p), half(vz_v, p)
        outp = half(out_v, p)

        def group_body(g, carry2):
            o = g * LANES
            osl = pl.ds(o, LANES)
            d = dp[osl]
            qj = qjp[osl]
            w1 = g1p[osl]
            w2 = g2p[osl]
            qi = lax.bitcast_convert_type(lax.shift_left(w1, 16), jnp.float32)
            dx = lax.bitcast_convert_type(jnp.bitwise_and(w1, -65536), jnp.float32)
            dy = lax.bitcast_convert_type(lax.shift_left(w2, 16), jnp.float32)
            dz = lax.bitcast_convert_type(jnp.bitwise_and(w2, -65536), jnp.float32)
            vx = vxp[osl]
            vy = vyp[osl]
            vz = vzp[osl]

            chi = 1.0 / d
            e1 = qi * qj * (chi - CHI_SHIFT)
            dot = (vx * dx + vy * dy + vz * dz) * chi
            e2 = qj * dot * (chi * chi - CHI2_SHIFT)
            x = jnp.clip((d - CUTON) * INV_W, 0.0, 1.0)
            sw = 1.0 - x * x * x * (x * (6.0 * x - 15.0) + 10.0)
            outp[osl] = KE * (e1 + e2) * sw
            return carry2

        lax.fori_loop(0, GROUPS, group_body, 0)

    def halfstep(k, p, first_pair, last_pair):
        gather_wait(p)
        if not last_pair or p == 0:
            instream_wait(k + 1, 1 - p)
            gather(1 - p)
        if not first_pair:
            outstream_wait(k - 2, p)
        compute(p)
        outstream(k, p)
        if not last_pair:
            instream(k + 2, p)

    instream(0, 0)
    instream_wait(0, 0)
    gather(0)
    instream(1, 1)

    halfstep(0, 0, True, False)
    halfstep(1, 1, True, False)

    def pair_body(kp, carry):
        k0 = 2 * kp
        halfstep(k0, 0, False, False)
        halfstep(k0 + 1, 1, False, False)
        return carry

    lax.fori_loop(1, NPAIR - 1, pair_body, 0)

    k0 = NCHUNK - 2
    halfstep(k0, 0, False, True)
    halfstep(k0 + 1, 1, False, True)

    outstream_wait(NCHUNK - 2, 0)
    outstream_wait(NCHUNK - 1, 1)


def kernel(mlmm_distances, atomic_charges, mlmm_atomic_charges,
           mlmm_idxu, mlmm_idxv, mlmm_vectors, atomic_dipoles):
    padn = PAD - N_ML

    def packw(a, b):
        a16 = lax.bitcast_convert_type(a.astype(jnp.bfloat16),
                                       jnp.uint16).astype(jnp.uint32)
        b16 = lax.bitcast_convert_type(b.astype(jnp.bfloat16),
                                       jnp.uint16).astype(jnp.uint32)
        return lax.bitcast_convert_type(a16 | (b16 << 16), jnp.int32)

    qdx = jnp.pad(packw(atomic_charges, atomic_dipoles[:, 0]), (0, padn))
    dydz = jnp.pad(packw(atomic_dipoles[:, 1], atomic_dipoles[:, 2]),
                   (0, padn))
    qmm = jnp.pad(mlmm_atomic_charges, (0, PAD - N_MM))
    vhalves = []
    for lo, hi in ((0, EHALF), (EHALF, E)):
        vhalves.append((mlmm_vectors[lo:hi, 0], mlmm_vectors[lo:hi, 1],
                        mlmm_vectors[lo:hi, 2]))

    def dbuf(dt=jnp.float32):
        return pltpu.VMEM((2 * CHUNK,), dt)

    import functools

    def mk(base0):
        return pl.kernel(
            functools.partial(_sc_body, base0=base0),
            out_type=jax.ShapeDtypeStruct((EHALF,), jnp.float32),
            mesh=plsc.VectorSubcoreMesh(core_axis_name="c",
                                        subcore_axis_name="s"),
            scratch_types=[
                pltpu.VMEM_SHARED((PAD,), jnp.int32),
                pltpu.VMEM_SHARED((PAD,), jnp.int32),
                pltpu.VMEM_SHARED((PAD,), jnp.float32),
                dbuf(), dbuf(jnp.int32), dbuf(jnp.int32), dbuf(), dbuf(),
                dbuf(),
                dbuf(jnp.int32), dbuf(jnp.int32), dbuf(),
                dbuf(),
                pltpu.SemaphoreType.DMA, pltpu.SemaphoreType.DMA,
                pltpu.SemaphoreType.DMA, pltpu.SemaphoreType.DMA,
                pltpu.SemaphoreType.DMA, pltpu.SemaphoreType.DMA,
            ],
        )

    outs = []
    for h, base0 in enumerate((0, EHALF)):
        fh = mk(base0)
        vx, vy, vz = vhalves[h]
        outs.append(fh(mlmm_distances, mlmm_idxu, mlmm_idxv, vx, vy, vz,
                       qdx, dydz, qmm))
    return jnp.concatenate(outs)


# speedup vs baseline: 1.1818x; 1.1818x over previous
import jax
import jax.numpy as jnp
from jax import lax
from jax.experimental import pallas as pl
from jax.experimental.pallas import tpu as pltpu
from jax.experimental.pallas import tpu_sc as plsc

E = 6_400_000
N_ML = 100_000
N_MM = 100_000

NC = 2
NS = 16
LANES = 16
NW = NC * NS
EPT = E // NW
CHUNK = 4000
NCHUNK = EPT // CHUNK
NPAIR = NCHUNK // 2
GROUPS = CHUNK // LANES

ROWS_PER_SUB = 6256
PAD = ROWS_PER_SUB * NS

CUTOFF = 12.0
KE = 332.0637
CUTON = 9.0
CHI_SHIFT = 1.0 / CUTOFF
CHI2_SHIFT = CHI_SHIFT * CHI_SHIFT
INV_W = 1.0 / (CUTOFF - CUTON)


def _sc_body(d_hbm, idxu_hbm, idxv_hbm, vx_hbm, vy_hbm, vz_hbm,
             qdx_hbm, dydz_hbm, qmm_hbm, out_hbm,
             qdx_s, dydz_s, qmm_s,
             d_v, idxu_v, idxv_v, vx_v, vy_v, vz_v,
             g1_v, g2_v, qj_v, out_v,
             sem_in0, sem_in1, sem_g0, sem_g1, sem_o0, sem_o1):
    c = lax.axis_index("c")
    s = lax.axis_index("s")
    wid = s * NC + c
    tile_base = wid * EPT

    sl = pl.ds(s * ROWS_PER_SUB, ROWS_PER_SUB)
    bounce_f = vx_v.at[pl.ds(0, ROWS_PER_SUB)]
    pltpu.sync_copy(qmm_hbm.at[sl], bounce_f)
    pltpu.sync_copy(bounce_f, qmm_s.at[sl])
    bounce_i = idxu_v.at[pl.ds(0, ROWS_PER_SUB)]
    for hbm_t, sp_t in ((qdx_hbm, qdx_s), (dydz_hbm, dydz_s)):
        pltpu.sync_copy(hbm_t.at[sl], bounce_i)
        pltpu.sync_copy(bounce_i, sp_t.at[sl])
    plsc.subcore_barrier()

    ins = (d_hbm, idxu_hbm, idxv_hbm, vx_hbm, vy_hbm, vz_hbm)
    inbufs = (d_v, idxu_v, idxv_v, vx_v, vy_v, vz_v)
    sem_in = (sem_in0, sem_in1)
    sem_g = (sem_g0, sem_g1)
    sem_o = (sem_o0, sem_o1)

    def ces(k):
        return pl.ds(tile_base + k * CHUNK, CHUNK)

    def half(buf, p):
        return buf.at[pl.ds(p * CHUNK, CHUNK)]

    def instream(k, p):
        sli = ces(k)
        for hbm_t, buf in zip(ins, inbufs):
            pltpu.async_copy(hbm_t.at[sli], half(buf, p), sem_in[p])

    def instream_wait(k, p):
        sli = ces(k)
        for hbm_t, buf in zip(ins, inbufs):
            pltpu.make_async_copy(hbm_t.at[sli], half(buf, p), sem_in[p]).wait()

    def gather(p):
        pltpu.async_copy(qdx_s.at[half(idxu_v, p)], half(g1_v, p), sem_g[p])
        pltpu.async_copy(dydz_s.at[half(idxu_v, p)], half(g2_v, p), sem_g[p])
        pltpu.async_copy(qmm_s.at[half(idxv_v, p)], half(qj_v, p), sem_g[p])

    def gather_wait(p):
        pltpu.make_async_copy(qdx_s.at[half(idxu_v, p)], half(g1_v, p),
                              sem_g[p]).wait()
        pltpu.make_async_copy(dydz_s.at[half(idxu_v, p)], half(g2_v, p),
                              sem_g[p]).wait()
        pltpu.make_async_copy(qmm_s.at[half(idxv_v, p)], half(qj_v, p),
                              sem_g[p]).wait()

    def outstream(k, p):
        pltpu.async_copy(half(out_v, p), out_hbm.at[ces(k)], sem_o[p])

    def outstream_wait(k, p):
        pltpu.make_async_copy(half(out_v, p), out_hbm.at[ces(k)],
                              sem_o[p]).wait()

    def compute(p):
        dp, qjp = half(d_v, p), half(qj_v, p)
        g1p, g2p = half(g1_v, p), half(g2_v, p)
        vxp, vyp, vzp = half(vx_v, p), half(vy_v, p), half(vz_v, p)
        outp = half(out_v, p)

        def group_body(g, carry2):
            o = g * LANES
            osl = pl.ds(o, LANES)
            d = dp[osl]
            qj = qjp[osl]
            w1 = g1p[osl]
            w2 = g2p[osl]
            qi = lax.bitcast_convert_type(lax.shift_left(w1, 16), jnp.float32)
            dx = lax.bitcast_convert_type(jnp.bitwise_and(w1, -65536), jnp.float32)
            dy = lax.bitcast_convert_type(lax.shift_left(w2, 16), jnp.float32)
            dz = lax.bitcast_convert_type(jnp.bitwise_and(w2, -65536), jnp.float32)
            vx = vxp[osl]
            vy = vyp[osl]
            vz = vzp[osl]

            chi = 1.0 / d
            e1 = qi * qj * (chi - CHI_SHIFT)
            dot = (vx * dx + vy * dy + vz * dz) * chi
            e2 = qj * dot * (chi * chi - CHI2_SHIFT)
            x = jnp.clip((d - CUTON) * INV_W, 0.0, 1.0)
            sw = 1.0 - x * x * x * (x * (6.0 * x - 15.0) + 10.0)
            outp[osl] = KE * (e1 + e2) * sw
            return carry2

        lax.fori_loop(0, GROUPS, group_body, 0)

    def halfstep(k, p, first_pair, last_pair):
        gather_wait(p)
        if not last_pair or p == 0:
            instream_wait(k + 1, 1 - p)
            gather(1 - p)
        if not first_pair:
            outstream_wait(k - 2, p)
        compute(p)
        outstream(k, p)
        if not last_pair:
            instream(k + 2, p)

    instream(0, 0)
    instream_wait(0, 0)
    gather(0)
    instream(1, 1)

    halfstep(0, 0, True, False)
    halfstep(1, 1, True, False)

    def pair_body(kp, carry):
        k0 = 2 * kp
        halfstep(k0, 0, False, False)
        halfstep(k0 + 1, 1, False, False)
        return carry

    lax.fori_loop(1, NPAIR - 1, pair_body, 0)

    k0 = NCHUNK - 2
    halfstep(k0, 0, False, True)
    halfstep(k0 + 1, 1, False, True)

    outstream_wait(NCHUNK - 2, 0)
    outstream_wait(NCHUNK - 1, 1)


def kernel(mlmm_distances, atomic_charges, mlmm_atomic_charges,
           mlmm_idxu, mlmm_idxv, mlmm_vectors, atomic_dipoles):
    padn = PAD - N_ML

    def packw(a, b):
        a16 = lax.bitcast_convert_type(a.astype(jnp.bfloat16),
                                       jnp.uint16).astype(jnp.uint32)
        b16 = lax.bitcast_convert_type(b.astype(jnp.bfloat16),
                                       jnp.uint16).astype(jnp.uint32)
        return lax.bitcast_convert_type(a16 | (b16 << 16), jnp.int32)

    qdx = jnp.pad(packw(atomic_charges, atomic_dipoles[:, 0]), (0, padn))
    dydz = jnp.pad(packw(atomic_dipoles[:, 1], atomic_dipoles[:, 2]),
                   (0, padn))
    qmm = jnp.pad(mlmm_atomic_charges, (0, PAD - N_MM))
    vx = mlmm_vectors[:, 0]
    vy = mlmm_vectors[:, 1]
    vz = mlmm_vectors[:, 2]

    def dbuf(dt=jnp.float32):
        return pltpu.VMEM((2 * CHUNK,), dt)

    f = pl.kernel(
        _sc_body,
        out_type=jax.ShapeDtypeStruct((E,), jnp.float32),
        mesh=plsc.VectorSubcoreMesh(core_axis_name="c", subcore_axis_name="s"),
        scratch_types=[
            pltpu.VMEM_SHARED((PAD,), jnp.int32),
            pltpu.VMEM_SHARED((PAD,), jnp.int32),
            pltpu.VMEM_SHARED((PAD,), jnp.float32),
            dbuf(), dbuf(jnp.int32), dbuf(jnp.int32), dbuf(), dbuf(), dbuf(),
            dbuf(jnp.int32), dbuf(jnp.int32), dbuf(),
            dbuf(),
            pltpu.SemaphoreType.DMA, pltpu.SemaphoreType.DMA,
            pltpu.SemaphoreType.DMA, pltpu.SemaphoreType.DMA,
            pltpu.SemaphoreType.DMA, pltpu.SemaphoreType.DMA,
        ],
    )
    return f(mlmm_distances, mlmm_idxu, mlmm_idxv, vx, vy, vz,
             qdx, dydz, qmm)
